# baseline (device time: 1556573 ns/iter reference)
import jax
import jax.numpy as jnp
from jax import lax
from jax.experimental import pallas as pl
from jax.experimental.pallas import tpu as pltpu

CHUNK = 1024
DEPTH = 2
NSLOT = 3


def kernel(x):
    m, n = x.shape
    n_chunks = m // CHUNK

    def body(x_send_ref, x_add_ref, out_ref, recv_ref, send_sems, recv_sems,
             credit_sem):
        i = pl.program_id(0)
        my_x = lax.axis_index("x")
        my_y = lax.axis_index("y")
        my_z = lax.axis_index("z")
        partner = (my_x, my_y, 1 - my_z)
        send_slot = i % NSLOT
        recv_slot = (i - DEPTH) % NSLOT

        @pl.when(i == 0)
        def _():
            barrier = pltpu.get_barrier_semaphore()
            pl.semaphore_signal(
                barrier,
                inc=1,
                device_id=partner,
                device_id_type=pl.DeviceIdType.MESH,
            )
            pl.semaphore_wait(barrier, 1)

        @pl.when((i >= NSLOT) & (i < n_chunks))
        def _():
            pl.semaphore_wait(credit_sem, 1)

        @pl.when(i < n_chunks)
        def _():
            send = pltpu.make_async_remote_copy(
                src_ref=x_send_ref,
                dst_ref=recv_ref.at[send_slot],
                send_sem=send_sems.at[send_slot],
                recv_sem=recv_sems.at[send_slot],
                device_id=partner,
                device_id_type=pl.DeviceIdType.MESH,
            )
            send.start()

        @pl.when(i >= DEPTH)
        def _():
            recv = pltpu.make_async_remote_copy(
                src_ref=x_send_ref,
                dst_ref=recv_ref.at[recv_slot],
                send_sem=send_sems.at[recv_slot],
                recv_sem=recv_sems.at[recv_slot],
                device_id=partner,
                device_id_type=pl.DeviceIdType.MESH,
            )
            recv.wait_recv()
            out_ref[...] = x_add_ref[...] + recv_ref[recv_slot]

        @pl.when((i >= DEPTH) & (i <= n_chunks + DEPTH - NSLOT - 1))
        def _():
            pl.semaphore_signal(
                credit_sem,
                inc=1,
                device_id=partner,
                device_id_type=pl.DeviceIdType.MESH,
            )

        @pl.when(i < n_chunks)
        def _():
            send = pltpu.make_async_remote_copy(
                src_ref=x_send_ref,
                dst_ref=recv_ref.at[send_slot],
                send_sem=send_sems.at[send_slot],
                recv_sem=recv_sems.at[send_slot],
                device_id=partner,
                device_id_type=pl.DeviceIdType.MESH,
            )
            send.wait_send()

    last = n_chunks - 1
    return pl.pallas_call(
        body,
        grid=(n_chunks + DEPTH,),
        in_specs=[
            pl.BlockSpec((CHUNK, n), lambda i: (jnp.minimum(i, last), 0)),
            pl.BlockSpec(
                (CHUNK, n), lambda i: (jnp.clip(i - DEPTH, 0, last), 0)
            ),
        ],
        out_specs=pl.BlockSpec(
            (CHUNK, n), lambda i: (jnp.clip(i - DEPTH, 0, last), 0)
        ),
        out_shape=jax.ShapeDtypeStruct((m, n), x.dtype),
        scratch_shapes=[
            pltpu.VMEM((NSLOT, CHUNK, n), x.dtype),
            pltpu.SemaphoreType.DMA((NSLOT,)),
            pltpu.SemaphoreType.DMA((NSLOT,)),
            pltpu.SemaphoreType.REGULAR,
        ],
        compiler_params=pltpu.CompilerParams(
            collective_id=0,
            vmem_limit_bytes=100 * 1024 * 1024,
        ),
    )(x, x)


# device time: 1536997 ns/iter; 1.0127x vs baseline; 1.0127x over previous
import jax
import jax.numpy as jnp
from jax import lax
from jax.experimental import pallas as pl
from jax.experimental.pallas import tpu as pltpu

CHUNK = 1024
DEPTH = 2
NSLOT = 3


def kernel(x):
    m, n = x.shape
    n_chunks = m // CHUNK

    def body(x_send_ref, x_add_ref, out_ref, recv_ref, send_sems, recv_sems,
             credit_sem):
        i = pl.program_id(0)
        my_x = lax.axis_index("x")
        my_y = lax.axis_index("y")
        my_z = lax.axis_index("z")
        partner = (my_x, my_y, 1 - my_z)
        send_slot = i % NSLOT
        recv_slot = (i - DEPTH) % NSLOT

        @pl.when(i == 0)
        def _():
            barrier = pltpu.get_barrier_semaphore()
            pl.semaphore_signal(
                barrier,
                inc=1,
                device_id=partner,
                device_id_type=pl.DeviceIdType.MESH,
            )
            pl.semaphore_wait(barrier, 1)

        @pl.when((i >= NSLOT) & (i < n_chunks))
        def _():
            pl.semaphore_wait(credit_sem, 1)

        @pl.when(i < n_chunks)
        def _():
            send = pltpu.make_async_remote_copy(
                src_ref=x_send_ref,
                dst_ref=recv_ref.at[send_slot],
                send_sem=send_sems.at[send_slot],
                recv_sem=recv_sems.at[send_slot],
                device_id=partner,
                device_id_type=pl.DeviceIdType.MESH,
            )
            send.start()

        @pl.when(i >= DEPTH)
        def _():
            recv = pltpu.make_async_remote_copy(
                src_ref=x_send_ref,
                dst_ref=recv_ref.at[recv_slot],
                send_sem=send_sems.at[recv_slot],
                recv_sem=recv_sems.at[recv_slot],
                device_id=partner,
                device_id_type=pl.DeviceIdType.MESH,
            )
            recv.wait_recv()
            out_ref[...] = x_add_ref[...] + recv_ref[recv_slot]

        @pl.when((i >= DEPTH) & (i <= n_chunks + DEPTH - NSLOT - 1))
        def _():
            pl.semaphore_signal(
                credit_sem,
                inc=1,
                device_id=partner,
                device_id_type=pl.DeviceIdType.MESH,
            )

        @pl.when((i >= 1) & (i <= n_chunks))
        def _():
            prev_send = pltpu.make_async_remote_copy(
                src_ref=x_send_ref,
                dst_ref=recv_ref.at[(i - 1) % NSLOT],
                send_sem=send_sems.at[(i - 1) % NSLOT],
                recv_sem=recv_sems.at[(i - 1) % NSLOT],
                device_id=partner,
                device_id_type=pl.DeviceIdType.MESH,
            )
            prev_send.wait_send()

    last = n_chunks - 1
    return pl.pallas_call(
        body,
        grid=(n_chunks + DEPTH,),
        in_specs=[
            pl.BlockSpec((CHUNK, n), lambda i: (jnp.minimum(i, last), 0)),
            pl.BlockSpec(
                (CHUNK, n), lambda i: (jnp.clip(i - DEPTH, 0, last), 0)
            ),
        ],
        out_specs=pl.BlockSpec(
            (CHUNK, n), lambda i: (jnp.clip(i - DEPTH, 0, last), 0)
        ),
        out_shape=jax.ShapeDtypeStruct((m, n), x.dtype),
        scratch_shapes=[
            pltpu.VMEM((NSLOT, CHUNK, n), x.dtype),
            pltpu.SemaphoreType.DMA((NSLOT,)),
            pltpu.SemaphoreType.DMA((NSLOT,)),
            pltpu.SemaphoreType.REGULAR,
        ],
        compiler_params=pltpu.CompilerParams(
            collective_id=0,
            vmem_limit_bytes=100 * 1024 * 1024,
        ),
    )(x, x)


# device time: 1536594 ns/iter; 1.0130x vs baseline; 1.0003x over previous
import jax
import jax.numpy as jnp
from jax import lax
from jax.experimental import pallas as pl
from jax.experimental.pallas import tpu as pltpu

CHUNK = 1024
NSLOT = 3


def kernel(x):
    m, n = x.shape
    n_chunks = m // CHUNK

    def body(x_ref, out_ref, stage_ref, recv_ref, copy_sems, send_sems,
             recv_sems, credit_sem):
        i = pl.program_id(0)
        my_x = lax.axis_index("x")
        my_y = lax.axis_index("y")
        my_z = lax.axis_index("z")
        partner = (my_x, my_y, 1 - my_z)

        @pl.when(i == 0)
        def _():
            barrier = pltpu.get_barrier_semaphore()
            pl.semaphore_signal(
                barrier,
                inc=1,
                device_id=partner,
                device_id_type=pl.DeviceIdType.MESH,
            )
            pl.semaphore_wait(barrier, 1)

        @pl.when((i >= NSLOT) & (i <= n_chunks + 1))
        def _():
            pltpu.make_async_remote_copy(
                src_ref=stage_ref.at[i % NSLOT],
                dst_ref=recv_ref.at[i % NSLOT],
                send_sem=send_sems.at[i % NSLOT],
                recv_sem=recv_sems.at[i % NSLOT],
                device_id=partner,
                device_id_type=pl.DeviceIdType.MESH,
            ).wait_send()

        @pl.when(i < n_chunks)
        def _():
            pltpu.make_async_copy(
                x_ref, stage_ref.at[i % NSLOT], copy_sems.at[i % NSLOT]
            ).start()

        @pl.when((i >= 1) & (i <= n_chunks))
        def _():
            pltpu.make_async_copy(
                x_ref, stage_ref.at[(i - 1) % NSLOT],
                copy_sems.at[(i - 1) % NSLOT],
            ).wait()

        @pl.when((i >= NSLOT + 1) & (i <= n_chunks))
        def _():
            pl.semaphore_wait(credit_sem, 1)

        @pl.when((i >= 1) & (i <= n_chunks))
        def _():
            pltpu.make_async_remote_copy(
                src_ref=stage_ref.at[(i - 1) % NSLOT],
                dst_ref=recv_ref.at[(i - 1) % NSLOT],
                send_sem=send_sems.at[(i - 1) % NSLOT],
                recv_sem=recv_sems.at[(i - 1) % NSLOT],
                device_id=partner,
                device_id_type=pl.DeviceIdType.MESH,
            ).start()

        @pl.when(i >= 2)
        def _():
            s = (i - 2) % NSLOT
            pltpu.make_async_remote_copy(
                src_ref=stage_ref.at[s],
                dst_ref=recv_ref.at[s],
                send_sem=send_sems.at[s],
                recv_sem=recv_sems.at[s],
                device_id=partner,
                device_id_type=pl.DeviceIdType.MESH,
            ).wait_recv()
            out_ref[...] = stage_ref[s] + recv_ref[s]

        @pl.when((i >= 2) & (i <= n_chunks - 2))
        def _():
            pl.semaphore_signal(
                credit_sem,
                inc=1,
                device_id=partner,
                device_id_type=pl.DeviceIdType.MESH,
            )

        @pl.when(i == n_chunks + 1)
        def _():
            s = (n_chunks - 1) % NSLOT
            pltpu.make_async_remote_copy(
                src_ref=stage_ref.at[s],
                dst_ref=recv_ref.at[s],
                send_sem=send_sems.at[s],
                recv_sem=recv_sems.at[s],
                device_id=partner,
                device_id_type=pl.DeviceIdType.MESH,
            ).wait_send()

    last = n_chunks - 1
    return pl.pallas_call(
        body,
        grid=(n_chunks + 2,),
        in_specs=[
            pl.BlockSpec((CHUNK, n), lambda i: (jnp.minimum(i, last), 0)),
        ],
        out_specs=pl.BlockSpec(
            (CHUNK, n), lambda i: (jnp.clip(i - 2, 0, last), 0)
        ),
        out_shape=jax.ShapeDtypeStruct((m, n), x.dtype),
        scratch_shapes=[
            pltpu.VMEM((NSLOT, CHUNK, n), x.dtype),
            pltpu.VMEM((NSLOT, CHUNK, n), x.dtype),
            pltpu.SemaphoreType.DMA((NSLOT,)),
            pltpu.SemaphoreType.DMA((NSLOT,)),
            pltpu.SemaphoreType.DMA((NSLOT,)),
            pltpu.SemaphoreType.REGULAR,
        ],
        compiler_params=pltpu.CompilerParams(
            collective_id=0,
            vmem_limit_bytes=100 * 1024 * 1024,
        ),
    )(x)
